# baseline (speedup 1.0000x reference)
import jax
import jax.numpy as jnp
from jax import lax
from jax.experimental import pallas as pl

D_MODEL = 768
DPART = 256
S_TOTAL = 4096
S_BLK = 4096
N_SBLK = S_TOTAL // S_BLK


def _body(ix_ref, iy_ref, iz_ref, x_ref, wx_ref, wy_ref, wz_ref, o_ref):
    iota = lax.broadcasted_iota(jnp.int32, (32, S_BLK), 0)

    def part(idx_ref, w_ref):
        oh = (idx_ref[0, 0, :][None, :] == iota).astype(jnp.float32)
        return lax.dot_general(
            oh, w_ref[...], (((0,), (0,)), ((), ())),
            preferred_element_type=jnp.float32,
        )

    ex = part(ix_ref, wx_ref)
    ey = part(iy_ref, wy_ref)
    ez = part(iz_ref, wz_ref)
    xb = x_ref[0]
    o_ref[0, :, 0:DPART] = xb[:, 0:DPART] + ex
    o_ref[0, :, DPART:2 * DPART] = xb[:, DPART:2 * DPART] + ey
    o_ref[0, :, 2 * DPART:D_MODEL] = xb[:, 2 * DPART:D_MODEL] + ez


def kernel(x, src_tgt, src_pos_x, src_pos_y, src_pos_z, Wx, Wy, Wz):
    del src_tgt
    B = x.shape[0]
    ix = src_pos_x.reshape(N_SBLK, 1, S_BLK)
    iy = src_pos_y.reshape(N_SBLK, 1, S_BLK)
    iz = src_pos_z.reshape(N_SBLK, 1, S_BLK)

    idx_spec = pl.BlockSpec((1, 1, S_BLK), lambda i, j: (i, 0, 0))
    tab_spec = pl.BlockSpec((32, DPART), lambda i, j: (0, 0))
    x_spec = pl.BlockSpec((1, S_BLK, D_MODEL), lambda i, j: (j, i, 0))

    return pl.pallas_call(
        _body,
        grid=(N_SBLK, B),
        in_specs=[idx_spec, idx_spec, idx_spec, x_spec, tab_spec, tab_spec,
                  tab_spec],
        out_specs=x_spec,
        out_shape=jax.ShapeDtypeStruct(x.shape, x.dtype),
    )(ix, iy, iz, x, Wx, Wy, Wz)
